# baseline (speedup 1.0000x reference)
import functools

import jax
import jax.numpy as jnp
from jax import lax
from jax.experimental import pallas as pl
from jax.experimental.pallas import tpu as pltpu
from jax.experimental.pallas import tpu_sc as plsc

N_FIELDS = 26
VOCAB = 100000
DIM = 32
BATCH = 16384

NC = 2
NS = 16
NW = NC * NS
TOTAL = BATCH * N_FIELDS
ROWS_PER_W = TOTAL // NW
CHUNK = 128
K = 8
GROUP = K * CHUNK
NCHUNK = ROWS_PER_W // CHUNK
NGROUP = ROWS_PER_W // GROUP


Q = 25088
RPF = Q
S = 3200


def _transpose_body(x_ref, y_ref):
    j = pl.program_id(1)
    eye = (lax.broadcasted_iota(jnp.int32, (DIM, DIM), 0) ==
           lax.broadcasted_iota(jnp.int32, (DIM, DIM), 1)).astype(jnp.float32)
    for jj in range(4):
        @pl.when(j == jj)
        def _():
            vbase = Q * jj
            vsize = min(Q, VOCAB - vbase)
            off = 0
            while off < vsize:
                sz = min(S, vsize - off)
                xj = x_ref[0, :, vbase + off:vbase + off + sz]
                c0 = jj * DIM
                y_ref[off:off + sz, c0:c0 + DIM] = (
                    lax.dot_general(xj, eye, (((0,), (0,)), ((), ())),
                                    preferred_element_type=jnp.float32))
                off += sz


def _tc_transpose(tab_t):
    return pl.pallas_call(
        _transpose_body,
        grid=(N_FIELDS, 4),
        in_specs=[pl.BlockSpec((1, DIM, VOCAB), lambda f, j: (f, 0, 0),
                               pipeline_mode=pl.Buffered(buffer_count=1))],
        out_specs=pl.BlockSpec((RPF, 128), lambda f, j: (f, 0),
                               pipeline_mode=pl.Buffered(buffer_count=1)),
        out_shape=jax.ShapeDtypeStruct((N_FIELDS * RPF, 128), jnp.float32),
    )(tab_t)


def _sc_gather(gidx, table_flat):
    mesh = plsc.VectorSubcoreMesh(core_axis_name="c", subcore_axis_name="s")

    @functools.partial(
        pl.kernel,
        out_type=jax.ShapeDtypeStruct((NW, NGROUP, K, CHUNK, DIM), jnp.float32),
        mesh=mesh,
        scratch_types=[
            pltpu.VMEM((NCHUNK, CHUNK), jnp.int32),
            pltpu.VMEM((2, K, CHUNK, DIM), jnp.float32),
            pltpu.SemaphoreType.DMA,
            pltpu.SemaphoreType.DMA,
        ],
        compiler_params=pltpu.CompilerParams(use_tc_tiling_on_sc=False),
    )
    def k(gidx_hbm, table_hbm, out_hbm, idx_v, buf, sem0, sem1):
        wid = lax.axis_index("s") * NC + lax.axis_index("c")
        sems = (sem0, sem1)
        pltpu.sync_copy(gidx_hbm.at[wid], idx_v)

        def fire(g):
            b = g % 2
            return [
                pltpu.async_copy(
                    table_hbm.at[idx_v.at[g * K + kk]], buf.at[b, kk], sems[b])
                for kk in range(K)
            ]

        handles = fire(0)
        for g in range(NGROUP):
            nxt = fire(g + 1) if g + 1 < NGROUP else []
            for h in handles:
                h.wait()
            pltpu.sync_copy(buf.at[g % 2], out_hbm.at[wid, g])
            handles = nxt

    return k(gidx, table_flat)


def kernel(observation, tables):
    offsets = (jnp.arange(N_FIELDS, dtype=jnp.int32) * (RPF * 4))[None, :]
    gidx = (offsets + (observation % Q) * 4 + observation // Q
            ).reshape(NW, NCHUNK, CHUNK)
    tab_t = tables.transpose(0, 2, 1)
    table_flat = _tc_transpose(tab_t).reshape(N_FIELDS * RPF * 4, DIM)
    out = _sc_gather(gidx, table_flat)
    return out.reshape(BATCH, N_FIELDS * DIM)


# speedup vs baseline: 1.1876x; 1.1876x over previous
import functools

import jax
import jax.numpy as jnp
from jax import lax
from jax.experimental import pallas as pl
from jax.experimental.pallas import tpu as pltpu
from jax.experimental.pallas import tpu_sc as plsc

N_FIELDS = 26
VOCAB = 100000
DIM = 32
BATCH = 16384

NC = 2
NS = 16
NW = NC * NS
TOTAL = BATCH * N_FIELDS
ROWS_PER_W = TOTAL // NW
CHUNK = 128
K = 8
GROUP = K * CHUNK
NCHUNK = ROWS_PER_W // CHUNK
NGROUP = ROWS_PER_W // GROUP


Q = 25088
RPF = Q
S = 3584


def _transpose_body(x_ref, y_ref):
    eye = (lax.broadcasted_iota(jnp.int32, (DIM, DIM), 0) ==
           lax.broadcasted_iota(jnp.int32, (DIM, DIM), 1)).astype(jnp.float32)

    def chunk(jj, off, sz):
        xj = x_ref[0, :, pl.ds(Q * jj + off, sz)]
        y_ref[pl.ds(off, sz), jj * DIM:(jj + 1) * DIM] = (
            lax.dot_general(xj, eye, (((0,), (0,)), ((), ())),
                            preferred_element_type=jnp.float32))

    for jj in range(4):
        nfull = 7 if jj < 3 else 6
        lax.fori_loop(
            0, nfull,
            lambda k, _: (chunk(jj, pl.multiple_of(k * S, 128), S), 0)[1], 0)
        if jj == 3:
            chunk(jj, 6 * S, VOCAB - 3 * Q - 6 * S)


def _tc_transpose(tab_t):
    return pl.pallas_call(
        _transpose_body,
        grid=(N_FIELDS,),
        in_specs=[pl.BlockSpec((1, DIM, VOCAB), lambda f: (f, 0, 0))],
        out_specs=pl.BlockSpec((RPF, 128), lambda f: (f, 0)),
        out_shape=jax.ShapeDtypeStruct((N_FIELDS * RPF, 128), jnp.float32),
    )(tab_t)


def _sc_gather(gidx, table_flat):
    mesh = plsc.VectorSubcoreMesh(core_axis_name="c", subcore_axis_name="s")

    @functools.partial(
        pl.kernel,
        out_type=jax.ShapeDtypeStruct((NW, NGROUP, K, CHUNK, DIM), jnp.float32),
        mesh=mesh,
        scratch_types=[
            pltpu.VMEM((NCHUNK, CHUNK), jnp.int32),
            pltpu.VMEM((2, K, CHUNK, DIM), jnp.float32),
            pltpu.SemaphoreType.DMA,
            pltpu.SemaphoreType.DMA,
        ],
        compiler_params=pltpu.CompilerParams(use_tc_tiling_on_sc=False),
    )
    def k(gidx_hbm, table_hbm, out_hbm, idx_v, buf, sem0, sem1):
        wid = lax.axis_index("s") * NC + lax.axis_index("c")
        sems = (sem0, sem1)
        pltpu.sync_copy(gidx_hbm.at[wid], idx_v)

        def fire(g):
            b = g % 2
            return [
                pltpu.async_copy(
                    table_hbm.at[idx_v.at[g * K + kk]], buf.at[b, kk], sems[b])
                for kk in range(K)
            ]

        handles = fire(0)
        for g in range(NGROUP):
            nxt = fire(g + 1) if g + 1 < NGROUP else []
            for h in handles:
                h.wait()
            pltpu.sync_copy(buf.at[g % 2], out_hbm.at[wid, g])
            handles = nxt

    return k(gidx, table_flat)


def kernel(observation, tables):
    offsets = (jnp.arange(N_FIELDS, dtype=jnp.int32) * (RPF * 4))[None, :]
    gidx = (offsets + (observation % Q) * 4 + observation // Q
            ).reshape(NW, NCHUNK, CHUNK)
    tab_t = tables.transpose(0, 2, 1)
    table_flat = _tc_transpose(tab_t).reshape(N_FIELDS * RPF * 4, DIM)
    out = _sc_gather(gidx, table_flat)
    return out.reshape(BATCH, N_FIELDS * DIM)


# speedup vs baseline: 2.6514x; 2.2325x over previous
import functools

import jax
import jax.numpy as jnp
from jax import lax
from jax.experimental import pallas as pl
from jax.experimental.pallas import tpu as pltpu
from jax.experimental.pallas import tpu_sc as plsc

N_FIELDS = 26
VOCAB = 100000
DIM = 32
BATCH = 16384

NC = 2
NS = 16
NW = NC * NS
TOTAL = BATCH * N_FIELDS
ROWS_PER_W = TOTAL // NW
CHUNK = 128
K = 8
GROUP = K * CHUNK
NCHUNK = ROWS_PER_W // CHUNK
NGROUP = ROWS_PER_W // GROUP


Q = 25088
RPF = Q
S = 3584


def _transpose_body(x_ref, y_ref):
    eye = (lax.broadcasted_iota(jnp.int32, (128, 128), 0) ==
           lax.broadcasted_iota(jnp.int32, (128, 128), 1)).astype(jnp.float32)

    def chunk(off, q3sz):
        parts = [x_ref[0, :, pl.ds(Q * jj + off, S)] for jj in range(3)]
        if q3sz == S:
            parts.append(x_ref[0, :, pl.ds(3 * Q + off, S)])
        else:
            parts.append(jnp.concatenate(
                [x_ref[0, :, pl.ds(3 * Q + off, q3sz)],
                 jnp.zeros((DIM, S - q3sz), jnp.float32)], axis=1))
        xcat = jnp.concatenate(parts, axis=0)
        y_ref[pl.ds(off, S), :] = lax.dot_general(
            xcat, eye, (((0,), (0,)), ((), ())),
            preferred_element_type=jnp.float32)

    nfull = 6
    lax.fori_loop(
        0, nfull,
        lambda k, _: (chunk(pl.multiple_of(k * S, 128), S), 0)[1], 0)
    chunk(6 * S, VOCAB - 3 * Q - 6 * S)


def _tc_transpose(tab_t):
    return pl.pallas_call(
        _transpose_body,
        grid=(N_FIELDS,),
        in_specs=[pl.BlockSpec((1, DIM, VOCAB), lambda f: (f, 0, 0))],
        out_specs=pl.BlockSpec((RPF, 128), lambda f: (f, 0)),
        out_shape=jax.ShapeDtypeStruct((N_FIELDS * RPF, 128), jnp.float32),
    )(tab_t)


def _sc_gather(gidx, table_flat):
    mesh = plsc.VectorSubcoreMesh(core_axis_name="c", subcore_axis_name="s")

    @functools.partial(
        pl.kernel,
        out_type=jax.ShapeDtypeStruct((NW, NGROUP, K, CHUNK, DIM), jnp.float32),
        mesh=mesh,
        scratch_types=[
            pltpu.VMEM((NCHUNK, CHUNK), jnp.int32),
            pltpu.VMEM((2, K, CHUNK, DIM), jnp.float32),
            pltpu.SemaphoreType.DMA,
            pltpu.SemaphoreType.DMA,
        ],
        compiler_params=pltpu.CompilerParams(use_tc_tiling_on_sc=False),
    )
    def k(gidx_hbm, table_hbm, out_hbm, idx_v, buf, sem0, sem1):
        wid = lax.axis_index("s") * NC + lax.axis_index("c")
        sems = (sem0, sem1)
        pltpu.sync_copy(gidx_hbm.at[wid], idx_v)

        def fire(g):
            b = g % 2
            return [
                pltpu.async_copy(
                    table_hbm.at[idx_v.at[g * K + kk]], buf.at[b, kk], sems[b])
                for kk in range(K)
            ]

        handles = fire(0)
        for g in range(NGROUP):
            nxt = fire(g + 1) if g + 1 < NGROUP else []
            for h in handles:
                h.wait()
            pltpu.sync_copy(buf.at[g % 2], out_hbm.at[wid, g])
            handles = nxt

    return k(gidx, table_flat)


def kernel(observation, tables):
    offsets = (jnp.arange(N_FIELDS, dtype=jnp.int32) * (RPF * 4))[None, :]
    gidx = (offsets + (observation % Q) * 4 + observation // Q
            ).reshape(NW, NCHUNK, CHUNK)
    tab_t = tables.transpose(0, 2, 1)
    table_flat = _tc_transpose(tab_t).reshape(N_FIELDS * RPF * 4, DIM)
    out = _sc_gather(gidx, table_flat)
    return out.reshape(BATCH, N_FIELDS * DIM)
